# baseline (speedup 1.0000x reference)
import functools

import jax
import jax.numpy as jnp
from jax import lax
from jax.experimental import pallas as pl
from jax.experimental.pallas import tpu as pltpu
from jax.experimental.pallas import tpu_sc as plsc

_N_NODES = 10000
_N_EDGES = 320000
_D = 128

_NC = 2
_NS = 16
_NW = _NC * _NS
_B_PER_W = _N_EDGES // _NW
_CHUNK = 192
_N_FULL = _B_PER_W // _CHUNK
_REM = _B_PER_W - _N_FULL * _CHUNK
_STAGE = 624
_N_CHUNKS = _B_PER_W // _CHUNK


def _mm_body(x_ref, w_ref, y_ref):
    y_ref[...] = lax.dot_general(
        x_ref[...], w_ref[...],
        dimension_numbers=(((1,), (1,)), ((), ())),
        preferred_element_type=jnp.float32)


def _linear(x, W):
    return pl.pallas_call(
        _mm_body,
        grid=(1,),
        in_specs=[pl.BlockSpec((_N_NODES, _D), lambda i: (0, 0)),
                  pl.BlockSpec((_D, _D), lambda i: (0, 0))],
        out_specs=pl.BlockSpec((_N_NODES, _D), lambda i: (0, 0)),
        out_shape=jax.ShapeDtypeStruct((_N_NODES, _D), jnp.float32),
    )(x, W)


_mesh = plsc.VectorSubcoreMesh(core_axis_name="c", subcore_axis_name="s")


@functools.partial(
    pl.kernel,
    mesh=_mesh,
    out_type=jax.ShapeDtypeStruct((_N_EDGES, _D), jnp.float32),
    scratch_types=[
        pltpu.VMEM((_CHUNK,), jnp.int32),
        pltpu.VMEM((_CHUNK,), jnp.int32),
        pltpu.VMEM((_CHUNK, _D), jnp.float32),
        pltpu.VMEM((_CHUNK, _D), jnp.float32),
        pltpu.VMEM_SHARED((_N_NODES, _D), jnp.float32),
        pltpu.SemaphoreType.DMA,
        pltpu.SemaphoreType.DMA,
        pltpu.SemaphoreType.DMA,
        pltpu.SemaphoreType.DMA,
        pltpu.SemaphoreType.DMA,
        pltpu.SemaphoreType.DMA,
    ],
)
def _gather_k(y_hbm, src_hbm, out_hbm, idx0, idx1, rows0, rows1, y_sp,
              ix0, ix1, in0, in1, out0, out1):
    wid = lax.axis_index("s") * _NC + lax.axis_index("c")
    base = wid * _B_PER_W
    sid = lax.axis_index("s")
    idx = (idx0, idx1)
    rows = (rows0, rows1)
    ix_sem = (ix0, ix1)
    in_sem = (in0, in1)
    out_sem = (out0, out1)

    def fire_idx(c, b, n=_CHUNK):
        return pltpu.async_copy(
            src_hbm.at[pl.ds(base + c * _CHUNK, n)],
            idx[b].at[pl.ds(0, n)], ix_sem[b])

    fire_idx(0, 0)
    fire_idx(1, 1)

    pltpu.sync_copy(y_hbm.at[pl.ds(sid * _STAGE, _STAGE)],
                    y_sp.at[pl.ds(sid * _STAGE, _STAGE)])

    @pl.when(sid == _NS - 1)
    def _():
        pltpu.sync_copy(y_hbm.at[pl.ds(_NS * _STAGE, _N_NODES - _NS * _STAGE)],
                        y_sp.at[pl.ds(_NS * _STAGE, _N_NODES - _NS * _STAGE)])

    plsc.subcore_barrier()

    def wait_idx(b, n=_CHUNK):
        pltpu.make_async_copy(
            src_hbm.at[pl.ds(base, n)],
            idx[b].at[pl.ds(0, n)], ix_sem[b]).wait()

    def fire_gather(c, b, n=_CHUNK):
        return pltpu.async_copy(
            y_sp.at[idx[b].at[pl.ds(0, n)]],
            rows[b].at[pl.ds(0, n)], in_sem[b])

    def wait_gather(b, n=_CHUNK):
        pltpu.make_async_copy(
            y_sp.at[idx[b].at[pl.ds(0, n)]],
            rows[b].at[pl.ds(0, n)], in_sem[b]).wait()

    def fire_writeback(c, b, n=_CHUNK):
        return pltpu.async_copy(
            rows[b].at[pl.ds(0, n)],
            out_hbm.at[pl.ds(base + c * _CHUNK, n)], out_sem[b])

    def wait_writeback(b, n=_CHUNK):
        pltpu.make_async_copy(
            rows[b].at[pl.ds(0, n)],
            out_hbm.at[pl.ds(base, n)], out_sem[b]).wait()

    wait_idx(0)
    fire_gather(0, 0)
    wait_idx(1)
    fire_gather(1, 1)
    wait_gather(0)
    fire_writeback(0, 0)
    fire_idx(2, 0)

    def pair_body(i, carry):
        c0 = 2 + 2 * i
        for b in range(2):
            c = c0 + b
            wait_idx(b)
            wait_writeback(b)
            fire_gather(c, b)
            wait_gather(1 - b)
            fire_writeback(c - 1, 1 - b)
            fire_idx(jnp.minimum(c + 1, _N_FULL - 1), 1 - b)
        return carry

    lax.fori_loop(0, (_N_FULL - 2) // 2, pair_body, 0)

    wait_gather(1)
    fire_writeback(_N_FULL - 1, 1)
    wait_idx(0)
    fire_idx(_N_FULL, 0, _REM)
    wait_idx(0, _REM)
    wait_writeback(0)
    fire_gather(_N_FULL, 0, _REM)
    wait_gather(0, _REM)
    fire_writeback(_N_FULL, 0, _REM)
    wait_writeback(1)
    wait_writeback(0, _REM)


def kernel(x, source, target, W):
    y = _linear(x, W)
    return _gather_k(y, source)


# speedup vs baseline: 1.0071x; 1.0071x over previous
import functools

import jax
import jax.numpy as jnp
from jax import lax
from jax.experimental import pallas as pl
from jax.experimental.pallas import tpu as pltpu
from jax.experimental.pallas import tpu_sc as plsc

_N_NODES = 10000
_N_EDGES = 320000
_D = 128

_NC = 2
_NS = 16
_NW = _NC * _NS
_B_PER_W = _N_EDGES // _NW
_CHUNK = 192
_N_FULL = _B_PER_W // _CHUNK
_REM = _B_PER_W - _N_FULL * _CHUNK
_STAGE = 624
_N_CHUNKS = _B_PER_W // _CHUNK


def _mm_body(x_ref, w_ref, y_ref):
    y_ref[...] = lax.dot_general(
        x_ref[...], w_ref[...],
        dimension_numbers=(((1,), (1,)), ((), ())),
        preferred_element_type=jnp.float32)


def _linear(x, W):
    return pl.pallas_call(
        _mm_body,
        grid=(2,),
        in_specs=[pl.BlockSpec((5000, _D), lambda i: (i, 0)),
                  pl.BlockSpec((_D, _D), lambda i: (0, 0))],
        out_specs=pl.BlockSpec((5000, _D), lambda i: (i, 0)),
        out_shape=jax.ShapeDtypeStruct((_N_NODES, _D), jnp.float32),
    )(x, W)


_mesh = plsc.VectorSubcoreMesh(core_axis_name="c", subcore_axis_name="s")


@functools.partial(
    pl.kernel,
    mesh=_mesh,
    out_type=jax.ShapeDtypeStruct((_N_EDGES, _D), jnp.float32),
    scratch_types=[
        pltpu.VMEM((_CHUNK,), jnp.int32),
        pltpu.VMEM((_CHUNK,), jnp.int32),
        pltpu.VMEM((_CHUNK, _D), jnp.float32),
        pltpu.VMEM((_CHUNK, _D), jnp.float32),
        pltpu.VMEM_SHARED((_N_NODES, _D), jnp.float32),
        pltpu.SemaphoreType.DMA,
        pltpu.SemaphoreType.DMA,
        pltpu.SemaphoreType.DMA,
        pltpu.SemaphoreType.DMA,
        pltpu.SemaphoreType.DMA,
        pltpu.SemaphoreType.DMA,
    ],
)
def _gather_k(y_hbm, src_hbm, out_hbm, idx0, idx1, rows0, rows1, y_sp,
              ix0, ix1, in0, in1, out0, out1):
    wid = lax.axis_index("s") * _NC + lax.axis_index("c")
    base = wid * _B_PER_W
    sid = lax.axis_index("s")
    idx = (idx0, idx1)
    rows = (rows0, rows1)
    ix_sem = (ix0, ix1)
    in_sem = (in0, in1)
    out_sem = (out0, out1)

    def fire_idx(c, b, n=_CHUNK):
        return pltpu.async_copy(
            src_hbm.at[pl.ds(base + c * _CHUNK, n)],
            idx[b].at[pl.ds(0, n)], ix_sem[b])

    fire_idx(0, 0)
    fire_idx(1, 1)

    pltpu.sync_copy(y_hbm.at[pl.ds(sid * _STAGE, _STAGE)],
                    y_sp.at[pl.ds(sid * _STAGE, _STAGE)])

    @pl.when(sid == _NS - 1)
    def _():
        pltpu.sync_copy(y_hbm.at[pl.ds(_NS * _STAGE, _N_NODES - _NS * _STAGE)],
                        y_sp.at[pl.ds(_NS * _STAGE, _N_NODES - _NS * _STAGE)])

    plsc.subcore_barrier()

    def wait_idx(b, n=_CHUNK):
        pltpu.make_async_copy(
            src_hbm.at[pl.ds(base, n)],
            idx[b].at[pl.ds(0, n)], ix_sem[b]).wait()

    def fire_gather(c, b, n=_CHUNK):
        return pltpu.async_copy(
            y_sp.at[idx[b].at[pl.ds(0, n)]],
            rows[b].at[pl.ds(0, n)], in_sem[b])

    def wait_gather(b, n=_CHUNK):
        pltpu.make_async_copy(
            y_sp.at[idx[b].at[pl.ds(0, n)]],
            rows[b].at[pl.ds(0, n)], in_sem[b]).wait()

    def fire_writeback(c, b, n=_CHUNK):
        return pltpu.async_copy(
            rows[b].at[pl.ds(0, n)],
            out_hbm.at[pl.ds(base + c * _CHUNK, n)], out_sem[b])

    def wait_writeback(b, n=_CHUNK):
        pltpu.make_async_copy(
            rows[b].at[pl.ds(0, n)],
            out_hbm.at[pl.ds(base, n)], out_sem[b]).wait()

    wait_idx(0)
    fire_gather(0, 0)
    wait_idx(1)
    fire_gather(1, 1)
    wait_gather(0)
    fire_writeback(0, 0)
    fire_idx(2, 0)

    def pair_body(i, carry):
        c0 = 2 + 2 * i
        for b in range(2):
            c = c0 + b
            wait_idx(b)
            wait_writeback(b)
            fire_gather(c, b)
            wait_gather(1 - b)
            fire_writeback(c - 1, 1 - b)
            fire_idx(jnp.minimum(c + 1, _N_FULL - 1), 1 - b)
        return carry

    lax.fori_loop(0, (_N_FULL - 2) // 2, pair_body, 0)

    wait_gather(1)
    fire_writeback(_N_FULL - 1, 1)
    wait_idx(0)
    fire_idx(_N_FULL, 0, _REM)
    wait_idx(0, _REM)
    wait_writeback(0)
    fire_gather(_N_FULL, 0, _REM)
    wait_gather(0, _REM)
    fire_writeback(_N_FULL, 0, _REM)
    wait_writeback(1)
    wait_writeback(0, _REM)


def kernel(x, source, target, W):
    y = _linear(x, W)
    return _gather_k(y, source)


# speedup vs baseline: 1.0075x; 1.0004x over previous
import functools

import jax
import jax.numpy as jnp
from jax import lax
from jax.experimental import pallas as pl
from jax.experimental.pallas import tpu as pltpu
from jax.experimental.pallas import tpu_sc as plsc

_N_NODES = 10000
_N_EDGES = 320000
_D = 128

_NC = 2
_NS = 16
_NW = _NC * _NS
_B_PER_W = _N_EDGES // _NW
_CHUNK = 192
_N_FULL = _B_PER_W // _CHUNK
_REM = _B_PER_W - _N_FULL * _CHUNK
_STAGE = 624


def _mm_body(x_ref, w_ref, y_ref):
    y_ref[...] = lax.dot_general(
        x_ref[...], w_ref[...],
        dimension_numbers=(((1,), (1,)), ((), ())),
        preferred_element_type=jnp.float32)


def _linear(x, W):
    return pl.pallas_call(
        _mm_body,
        grid=(2,),
        in_specs=[pl.BlockSpec((5000, _D), lambda i: (i, 0)),
                  pl.BlockSpec((_D, _D), lambda i: (0, 0))],
        out_specs=pl.BlockSpec((5000, _D), lambda i: (i, 0)),
        out_shape=jax.ShapeDtypeStruct((_N_NODES, _D), jnp.float32),
    )(x, W)


_mesh = plsc.VectorSubcoreMesh(core_axis_name="c", subcore_axis_name="s")


@functools.partial(
    pl.kernel,
    mesh=_mesh,
    out_type=jax.ShapeDtypeStruct((_N_EDGES, _D), jnp.float32),
    scratch_types=[
        pltpu.VMEM((_CHUNK,), jnp.int32),
        pltpu.VMEM((_CHUNK,), jnp.int32),
        pltpu.VMEM((_CHUNK, _D), jnp.float32),
        pltpu.VMEM((_CHUNK, _D), jnp.float32),
        pltpu.VMEM_SHARED((_N_NODES, _D), jnp.float32),
        pltpu.SemaphoreType.DMA,
        pltpu.SemaphoreType.DMA,
        pltpu.SemaphoreType.DMA,
        pltpu.SemaphoreType.DMA,
        pltpu.SemaphoreType.DMA,
        pltpu.SemaphoreType.DMA,
    ],
)
def _gather_k(y_hbm, src_hbm, out_hbm, idx0, idx1, rows0, rows1, y_sp,
              ix0, ix1, in0, in1, out0, out1):
    wid = lax.axis_index("s") * _NC + lax.axis_index("c")
    base = wid * _B_PER_W
    sid = lax.axis_index("s")
    idx = (idx0, idx1)
    rows = (rows0, rows1)
    ix_sem = (ix0, ix1)
    in_sem = (in0, in1)
    out_sem = (out0, out1)

    def fire_idx(c, b, n=_CHUNK):
        return pltpu.async_copy(
            src_hbm.at[pl.ds(base + c * _CHUNK, n)],
            idx[b].at[pl.ds(0, n)], ix_sem[b])

    fire_idx(0, 0)
    fire_idx(1, 1)

    pltpu.sync_copy(y_hbm.at[pl.ds(sid * _STAGE, _STAGE)],
                    y_sp.at[pl.ds(sid * _STAGE, _STAGE)])

    @pl.when(sid == _NS - 1)
    def _():
        pltpu.sync_copy(y_hbm.at[pl.ds(_NS * _STAGE, _N_NODES - _NS * _STAGE)],
                        y_sp.at[pl.ds(_NS * _STAGE, _N_NODES - _NS * _STAGE)])

    plsc.subcore_barrier()

    def wait_idx(b, n=_CHUNK):
        pltpu.make_async_copy(
            src_hbm.at[pl.ds(base, n)],
            idx[b].at[pl.ds(0, n)], ix_sem[b]).wait()

    def fire_gather(c, b, n=_CHUNK):
        return pltpu.async_copy(
            y_sp.at[idx[b].at[pl.ds(0, n)]],
            rows[b].at[pl.ds(0, n)], in_sem[b])

    def wait_gather(b, n=_CHUNK):
        pltpu.make_async_copy(
            y_sp.at[idx[b].at[pl.ds(0, n)]],
            rows[b].at[pl.ds(0, n)], in_sem[b]).wait()

    def fire_writeback(c, b, n=_CHUNK):
        return pltpu.async_copy(
            rows[b].at[pl.ds(0, n)],
            out_hbm.at[pl.ds(base + c * _CHUNK, n)], out_sem[b])

    def wait_writeback(b, n=_CHUNK):
        pltpu.make_async_copy(
            rows[b].at[pl.ds(0, n)],
            out_hbm.at[pl.ds(base, n)], out_sem[b]).wait()

    wait_idx(0)
    fire_gather(0, 0)
    wait_idx(1)
    fire_gather(1, 1)
    wait_gather(0)
    fire_writeback(0, 0)
    fire_idx(2, 0)

    def pair_body(i, carry):
        c0 = 2 + 2 * i
        for b in range(2):
            c = c0 + b
            wait_idx(b)
            wait_writeback(b)
            fire_gather(c, b)
            wait_gather(1 - b)
            fire_writeback(c - 1, 1 - b)
            fire_idx(jnp.minimum(c + 1, _N_FULL - 1), 1 - b)
        return carry

    lax.fori_loop(0, (_N_FULL - 2) // 2, pair_body, 0)

    wait_gather(1)
    fire_writeback(_N_FULL - 1, 1)
    wait_idx(0)
    fire_idx(_N_FULL, 0, _REM)
    wait_idx(0, _REM)
    wait_writeback(0)
    fire_gather(_N_FULL, 0, _REM)
    wait_gather(0, _REM)
    fire_writeback(_N_FULL, 0, _REM)
    wait_writeback(1)
    wait_writeback(0, _REM)


def kernel(x, source, target, W):
    y = _linear(x, W)
    return _gather_k(y, source)
